# Initial kernel scaffold; baseline (speedup 1.0000x reference)
import functools

import jax
import jax.numpy as jnp
from jax import lax
from jax.experimental import pallas as pl
from jax.experimental.pallas import tpu as pltpu
from jax.experimental.pallas import tpu_sc as plsc

_N = 100000
_E = 6400000
_G = 128
_D = 16
_GRP = 128
_NGRP = _E // _GRP
_NSUB = 16
_NCORE = 2
_NW = _NSUB * _NCORE
_STRIPE = _N // _NSUB
_R = 1000
_NBLK = _N // _R


def _make_edge_pass(split_features: bool):
  mesh = plsc.VectorSubcoreMesh(core_axis_name="c", subcore_axis_name="s")

  @functools.partial(
      pl.kernel,
      out_type=jax.ShapeDtypeStruct((_NCORE, _N, _D), jnp.float32),
      mesh=mesh,
      scratch_types=[
          pltpu.VMEM((2, _GRP), jnp.int32),
          pltpu.VMEM((2, _GRP), jnp.int32),
          pltpu.VMEM((2, _GRP), jnp.float32),
          pltpu.VMEM((2, _GRP, _D), jnp.float32),
          pltpu.VMEM_SHARED((_N, _D), jnp.float32),
          pltpu.SemaphoreType.DMA,
          pltpu.SemaphoreType.DMA,
          pltpu.SemaphoreType.DMA,
          pltpu.SemaphoreType.DMA,
          pltpu.SemaphoreType.DMA,
          pltpu.SemaphoreType.DMA,
      ],
  )
  def edge_pass(tbl_a, tbl_b, edges, ew2d, zeros, out,
                sidx, didx, ewv, rows, acc,
                lsem_a, lsem_b, gsem_a, gsem_b, ssem_a, ssem_b):
    c = lax.axis_index("c")
    s = lax.axis_index("s")
    lsem = (lsem_a, lsem_b)
    gsem = (gsem_a, gsem_b)
    ssem = (ssem_a, ssem_b)

    pltpu.sync_copy(zeros, acc.at[pl.ds(s * _STRIPE, _STRIPE)])
    plsc.subcore_barrier()

    if split_features:
      g0 = s * (_NGRP // _NSUB)
      ng = jnp.int32(_NGRP // _NSUB)
    else:
      wid = c * _NSUB + s
      per = _NGRP // _NW
      rem = _NGRP - _NW * per
      g0 = wid * per + jnp.minimum(wid, rem)
      ng = jnp.int32(per) + (wid < rem).astype(jnp.int32)

    def fire_idx(b, g):
      pltpu.async_copy(edges.at[0, g], sidx.at[b], lsem[b])
      pltpu.async_copy(edges.at[1, g], didx.at[b], lsem[b])
      pltpu.async_copy(ew2d.at[g], ewv.at[b], lsem[b])

    def wait_idx(b):
      pltpu.make_async_copy(edges.at[0, 0], sidx.at[b], lsem[b]).wait()
      pltpu.make_async_copy(edges.at[1, 0], didx.at[b], lsem[b]).wait()
      pltpu.make_async_copy(ew2d.at[0], ewv.at[b], lsem[b]).wait()

    def fire_gather(b):
      @pl.when(c == 0)
      def _():
        pltpu.async_copy(tbl_a.at[sidx.at[b]], rows.at[b], gsem[b])

      @pl.when(c == 1)
      def _():
        pltpu.async_copy(tbl_b.at[sidx.at[b]], rows.at[b], gsem[b])

    def wait_gather(b):
      pltpu.make_async_copy(tbl_a.at[sidx.at[b]], rows.at[b], gsem[b]).wait()

    def fire_scatter(b):
      pltpu.async_copy(rows.at[b], acc.at[didx.at[b]], ssem[b], add=True)

    def wait_scatter(b):
      pltpu.make_async_copy(rows.at[b], acc.at[didx.at[b]], ssem[b]).wait()

    def scale(b):
      @plsc.parallel_loop(0, _GRP, step=1, unroll=8)
      def _(r):
        rows[b, r] = rows[b, r] * ewv[b, r]

    def do_group(b):
      wait_idx(b)
      fire_gather(b)
      wait_gather(b)
      scale(b)
      fire_scatter(b)

    npairs = ng // 2

    def chunk(ch, carry):
      i0 = g0 + 2 * ch

      @pl.when(ch > 0)
      def _():
        wait_scatter(0)
        wait_scatter(1)

      fire_idx(0, i0)
      fire_idx(1, i0 + 1)
      do_group(0)
      do_group(1)
      return carry

    lax.fori_loop(0, npairs, chunk, 0)

    @pl.when(npairs > 0)
    def _():
      wait_scatter(0)
      wait_scatter(1)

    @pl.when(ng % 2 == 1)
    def _():
      g = g0 + ng - 1
      fire_idx(0, g)
      do_group(0)
      wait_scatter(0)

    plsc.subcore_barrier()
    pltpu.sync_copy(acc.at[pl.ds(s * _STRIPE, _STRIPE)],
                    out.at[c, pl.ds(s * _STRIPE, _STRIPE)])

  return edge_pass


_edge_pass_half = _make_edge_pass(False)
_edge_pass_feat = _make_edge_pass(True)


def _t1_body(x_ref, a_ref, o_ref):
  o_ref[...] = jnp.dot(x_ref[...], a_ref[...],
                       preferred_element_type=jnp.float32)


def _tc_pretable(x, a1):
  return pl.pallas_call(
      _t1_body,
      grid=(_NBLK,),
      in_specs=[pl.BlockSpec((_R, 4), lambda i: (i, 0)),
                pl.BlockSpec((4, _D), lambda i: (0, 0))],
      out_specs=pl.BlockSpec((_R, _D), lambda i: (i, 0)),
      out_shape=jax.ShapeDtypeStruct((_N, _D), jnp.float32),
  )(x, a1)


def _t2_body(p_ref, x_ref, r1_ref, c1_ref, o_ref):
  agg = p_ref[0] + p_ref[1]
  z = agg + jnp.dot(x_ref[...], r1_ref[...],
                    preferred_element_type=jnp.float32) + c1_ref[...]
  o_ref[...] = jnp.maximum(z, 0.0)


def _tc_layer1(parts1, x, r1, c1):
  return pl.pallas_call(
      _t2_body,
      grid=(_NBLK,),
      in_specs=[pl.BlockSpec((2, _R, _D), lambda i: (0, i, 0)),
                pl.BlockSpec((_R, 4), lambda i: (i, 0)),
                pl.BlockSpec((4, _D), lambda i: (0, 0)),
                pl.BlockSpec((1, _D), lambda i: (0, 0))],
      out_specs=pl.BlockSpec((_R, _D), lambda i: (i, 0)),
      out_shape=jax.ShapeDtypeStruct((_N, _D), jnp.float32),
  )(parts1, x, r1, c1)


def _t3_body(p_ref, h1_ref, a2_ref, r2_ref, c2_ref, oa_ref, ob_ref):
  agg = p_ref[0] + p_ref[1]
  z = (jnp.dot(agg, a2_ref[...], preferred_element_type=jnp.float32)
       + jnp.dot(h1_ref[...], r2_ref[...], preferred_element_type=jnp.float32)
       + c2_ref[...])
  h2 = jnp.maximum(z, 0.0)
  oa_ref[...] = h2[:, :_D]
  ob_ref[...] = h2[:, _D:]


def _tc_layer2(parts2, h1, a2, r2, c2):
  return pl.pallas_call(
      _t3_body,
      grid=(_NBLK,),
      in_specs=[pl.BlockSpec((2, _R, _D), lambda i: (0, i, 0)),
                pl.BlockSpec((_R, _D), lambda i: (i, 0)),
                pl.BlockSpec((_D, 32), lambda i: (0, 0)),
                pl.BlockSpec((_D, 32), lambda i: (0, 0)),
                pl.BlockSpec((1, 32), lambda i: (0, 0))],
      out_specs=[pl.BlockSpec((_R, _D), lambda i: (i, 0)),
                 pl.BlockSpec((_R, _D), lambda i: (i, 0))],
      out_shape=[jax.ShapeDtypeStruct((_N, _D), jnp.float32),
                 jax.ShapeDtypeStruct((_N, _D), jnp.float32)],
  )(parts2, h1, a2, r2, c2)


def _t4_body(p_ref, ha_ref, hb_ref, a3_ref, r3_ref, c3_ref, b_ref,
             sums_ref, cnts_ref):
  agg = jnp.concatenate([p_ref[0], p_ref[1]], axis=1)
  h2 = jnp.concatenate([ha_ref[...], hb_ref[...]], axis=1)
  z = (jnp.dot(agg, a3_ref[...], preferred_element_type=jnp.float32)
       + jnp.dot(h2, r3_ref[...], preferred_element_type=jnp.float32)
       + c3_ref[...])
  h3 = jnp.maximum(z, 0.0)
  bb = b_ref[0, 0, :]
  oh = (bb[:, None] == lax.broadcasted_iota(jnp.int32, (_R, _G), 1)
        ).astype(jnp.float32)

  @pl.when(pl.program_id(0) == 0)
  def _():
    sums_ref[...] = jnp.zeros_like(sums_ref)
    cnts_ref[...] = jnp.zeros_like(cnts_ref)

  sums_ref[...] += lax.dot_general(oh, h3, (((0,), (0,)), ((), ())),
                                   preferred_element_type=jnp.float32)
  cnts_ref[...] += jnp.sum(oh, axis=0)[None, :]


def _tc_layer3_pool(parts3, h2a, h2b, a3, r3, c3, batch3):
  return pl.pallas_call(
      _t4_body,
      grid=(_NBLK,),
      in_specs=[pl.BlockSpec((2, _R, _D), lambda i: (0, i, 0)),
                pl.BlockSpec((_R, _D), lambda i: (i, 0)),
                pl.BlockSpec((_R, _D), lambda i: (i, 0)),
                pl.BlockSpec((32, 64), lambda i: (0, 0)),
                pl.BlockSpec((32, 64), lambda i: (0, 0)),
                pl.BlockSpec((1, 64), lambda i: (0, 0)),
                pl.BlockSpec((1, 1, _R), lambda i: (i, 0, 0))],
      out_specs=[pl.BlockSpec((_G, 64), lambda i: (0, 0)),
                 pl.BlockSpec((1, _G), lambda i: (0, 0))],
      out_shape=[jax.ShapeDtypeStruct((_G, 64), jnp.float32),
                 jax.ShapeDtypeStruct((1, _G), jnp.float32)],
  )(parts3, h2a, h2b, a3, r3, c3, batch3)


def _t5_body(sums_ref, cnts_ref, f1_ref, cf_ref, f4_ref, b4_ref, o_ref):
  cnt = jnp.maximum(cnts_ref[0, :], 1.0)
  pooled = sums_ref[...] / cnt[:, None]
  z = jnp.maximum(jnp.dot(pooled, f1_ref[...],
                          preferred_element_type=jnp.float32) + cf_ref[...],
                  0.0)
  o = jnp.dot(z, f4_ref[...], preferred_element_type=jnp.float32) + b4_ref[...]
  m = jnp.max(o, axis=1, keepdims=True)
  lse = m + jnp.log(jnp.sum(jnp.exp(o - m), axis=1, keepdims=True))
  o_ref[...] = o - lse


def _tc_head(sums, cnts, f1, cf, f4, b4):
  return pl.pallas_call(
      _t5_body,
      in_specs=[pl.BlockSpec((_G, 64), lambda: (0, 0)),
                pl.BlockSpec((1, _G), lambda: (0, 0)),
                pl.BlockSpec((64, _D), lambda: (0, 0)),
                pl.BlockSpec((1, _D), lambda: (0, 0)),
                pl.BlockSpec((_D, 2), lambda: (0, 0)),
                pl.BlockSpec((1, 2), lambda: (0, 0))],
      out_specs=pl.BlockSpec((_G, 2), lambda: (0, 0)),
      out_shape=jax.ShapeDtypeStruct((_G, 2), jnp.float32),
  )(sums, cnts, f1, cf, f4, b4)


def kernel(x, edge_index, edge_weight, edge_attr, batch,
           W1_rel, b1_rel, W1_root, W2_rel, b2_rel, W2_root,
           W3_rel, b3_rel, W3_root, Wfc1, bfc1, Wfc4, bfc4,
           bn1_g, bn1_b, bn1_rm, bn1_rv,
           bn2_g, bn2_b, bn2_rm, bn2_rv,
           bn3_g, bn3_b, bn3_rm, bn3_rv,
           bnf_g, bnf_b, bnf_rm, bnf_rv):
  eps = 1e-5
  s1 = bn1_g / jnp.sqrt(bn1_rv + eps)
  t1 = bn1_b - bn1_rm * s1
  s2 = bn2_g / jnp.sqrt(bn2_rv + eps)
  t2 = bn2_b - bn2_rm * s2
  s3 = bn3_g / jnp.sqrt(bn3_rv + eps)
  t3 = bn3_b - bn3_rm * s3
  sf = bnf_g / jnp.sqrt(bnf_rv + eps)
  tf = bnf_b - bnf_rm * sf

  a1 = W1_rel.T * s1[None, :]
  r1 = W1_root.T * s1[None, :]
  c1 = (b1_rel * s1 + t1)[None, :]
  a2 = W2_rel.T * s2[None, :]
  r2 = W2_root.T * s2[None, :]
  c2 = (b2_rel * s2 + t2)[None, :]
  a3 = W3_rel.T * s3[None, :]
  r3 = W3_root.T * s3[None, :]
  c3 = (b3_rel * s3 + t3)[None, :]
  f1 = Wfc1.T * sf[None, :]
  cf = (bfc1 * sf + tf)[None, :]
  f4 = Wfc4.T
  b4 = bfc4[None, :]

  edges3 = edge_index.reshape(2, _NGRP, _GRP)
  ew2 = edge_weight.reshape(_NGRP, _GRP)
  zeros = jnp.zeros((_STRIPE, _D), jnp.float32)
  batch3 = batch.reshape(_NBLK, 1, _R)

  y1 = _tc_pretable(x, a1)
  parts1 = _edge_pass_half(y1, y1, edges3, ew2, zeros)
  h1 = _tc_layer1(parts1, x, r1, c1)
  parts2 = _edge_pass_half(h1, h1, edges3, ew2, zeros)
  h2a, h2b = _tc_layer2(parts2, h1, a2, r2, c2)
  parts3 = _edge_pass_feat(h2a, h2b, edges3, ew2, zeros)
  sums, cnts = _tc_layer3_pool(parts3, h2a, h2b, a3, r3, c3, batch3)
  return _tc_head(sums, cnts, f1, cf, f4, b4)


# speedup vs baseline: 29.0794x; 29.0794x over previous
import functools

import jax
import jax.numpy as jnp
from jax import lax
from jax.experimental import pallas as pl
from jax.experimental.pallas import tpu as pltpu
from jax.experimental.pallas import tpu_sc as plsc

_N = 100000
_E = 6400000
_G = 128
_D = 16
_GRP = 128
_SG = 4
_CH = _SG * _GRP
_NCHUNK = _E // _CH
_NSUB = 16
_NCORE = 2
_NW = _NSUB * _NCORE
_STRIPE = 6256
_STRIPE_LAST = _N - 15 * _STRIPE
_R = 1000
_NBLK = _N // _R


def _make_edge_pass(split_features: bool):
  mesh = plsc.VectorSubcoreMesh(core_axis_name="c", subcore_axis_name="s")

  @functools.partial(
      pl.kernel,
      out_type=jax.ShapeDtypeStruct((_NCORE, _N, _D), jnp.float32),
      mesh=mesh,
      compiler_params=pltpu.CompilerParams(use_tc_tiling_on_sc=False),
      scratch_types=[
          pltpu.VMEM((2, _SG, _GRP), jnp.int32),
          pltpu.VMEM((2, _SG, _GRP), jnp.int32),
          pltpu.VMEM((2, _SG, _GRP), jnp.float32),
          pltpu.VMEM((2, _CH, _D), jnp.float32),
          pltpu.VMEM_SHARED((_N, _D), jnp.float32),
          pltpu.SemaphoreType.DMA,
          pltpu.SemaphoreType.DMA,
          pltpu.SemaphoreType.DMA,
          pltpu.SemaphoreType.DMA,
          pltpu.SemaphoreType.DMA,
          pltpu.SemaphoreType.DMA,
      ],
  )
  def edge_pass(tbl_a, tbl_b, edges, ew2d, zeros, out,
                sidx, didx, ewv, rows, acc,
                lsem_a, lsem_b, gsem_a, gsem_b, ssem_a, ssem_b):
    c = lax.axis_index("c")
    s = lax.axis_index("s")
    lsem = (lsem_a, lsem_b)
    gsem = (gsem_a, gsem_b)
    ssem = (ssem_a, ssem_b)

    @pl.when(s < _NSUB - 1)
    def _():
      pltpu.sync_copy(zeros, acc.at[pl.ds(s * _STRIPE, _STRIPE)])

    @pl.when(s == _NSUB - 1)
    def _():
      pltpu.sync_copy(zeros.at[pl.ds(0, _STRIPE_LAST)],
                      acc.at[pl.ds((_NSUB - 1) * _STRIPE, _STRIPE_LAST)])

    plsc.subcore_barrier()

    if split_features:
      nw, w = _NSUB, s
    else:
      nw, w = _NW, c * _NSUB + s
    per = _NCHUNK // nw
    rem = _NCHUNK % nw
    g0 = w * per + jnp.minimum(w, rem)
    ng = jnp.int32(per) + (w < rem).astype(jnp.int32)

    def fire_idx(b, g):
      pltpu.async_copy(edges.at[0, g], sidx.at[b], lsem[b])
      pltpu.async_copy(edges.at[1, g], didx.at[b], lsem[b])
      pltpu.async_copy(ew2d.at[g], ewv.at[b], lsem[b])

    def wait_idx(b):
      pltpu.make_async_copy(edges.at[0, 0], sidx.at[b], lsem[b]).wait()
      pltpu.make_async_copy(edges.at[1, 0], didx.at[b], lsem[b]).wait()
      pltpu.make_async_copy(ew2d.at[0], ewv.at[b], lsem[b]).wait()

    def fire_gather(b):
      @pl.when(c == 0)
      def _():
        for j in range(_SG):
          pltpu.async_copy(tbl_a.at[sidx.at[b, j]],
                           rows.at[b, pl.ds(j * _GRP, _GRP)], gsem[b])

      @pl.when(c == 1)
      def _():
        for j in range(_SG):
          pltpu.async_copy(tbl_b.at[sidx.at[b, j]],
                           rows.at[b, pl.ds(j * _GRP, _GRP)], gsem[b])

    def wait_gather(b):
      for j in range(_SG):
        pltpu.make_async_copy(tbl_a.at[sidx.at[b, j]],
                              rows.at[b, pl.ds(j * _GRP, _GRP)],
                              gsem[b]).wait()

    def fire_scatter(b):
      for j in range(_SG):
        pltpu.async_copy(rows.at[b, pl.ds(j * _GRP, _GRP)],
                         acc.at[didx.at[b, j]], ssem[b], add=True)

    def wait_scatter(b):
      for j in range(_SG):
        pltpu.make_async_copy(rows.at[b, pl.ds(j * _GRP, _GRP)],
                              acc.at[didx.at[b, j]], ssem[b]).wait()

    def scale(b):
      @plsc.parallel_loop(0, _SG, step=1)
      def _(j):
        for t in range(_GRP // 16):
          wv = ewv[b, j, pl.ds(t * 16, 16)]
          for k in range(16):
            r = j * _GRP + t * 16 + k
            rows[b, r] = rows[b, r] * wv[k]

    def do_chunk(b):
      wait_idx(b)
      fire_gather(b)
      wait_gather(b)
      scale(b)
      fire_scatter(b)

    npairs = ng // 2

    def chunk(ch, carry):
      i0 = g0 + 2 * ch

      @pl.when(ch > 0)
      def _():
        wait_scatter(0)
        wait_scatter(1)

      fire_idx(0, i0)
      fire_idx(1, i0 + 1)
      do_chunk(0)
      do_chunk(1)
      return carry

    lax.fori_loop(0, npairs, chunk, 0)

    @pl.when(npairs > 0)
    def _():
      wait_scatter(0)
      wait_scatter(1)

    @pl.when(ng % 2 == 1)
    def _():
      g = g0 + ng - 1
      fire_idx(0, g)
      do_chunk(0)
      wait_scatter(0)

    plsc.subcore_barrier()

    @pl.when(s < _NSUB - 1)
    def _():
      pltpu.sync_copy(acc.at[pl.ds(s * _STRIPE, _STRIPE)],
                      out.at[c, pl.ds(s * _STRIPE, _STRIPE)])

    @pl.when(s == _NSUB - 1)
    def _():
      pltpu.sync_copy(acc.at[pl.ds((_NSUB - 1) * _STRIPE, _STRIPE_LAST)],
                      out.at[c, pl.ds((_NSUB - 1) * _STRIPE, _STRIPE_LAST)])

  return edge_pass


_edge_pass_half = _make_edge_pass(False)
_edge_pass_feat = _make_edge_pass(True)


def _t1_body(x_ref, a_ref, o_ref):
  o_ref[...] = jnp.dot(x_ref[...], a_ref[...],
                       preferred_element_type=jnp.float32)


def _tc_pretable(x, a1):
  return pl.pallas_call(
      _t1_body,
      grid=(_NBLK,),
      in_specs=[pl.BlockSpec((_R, 4), lambda i: (i, 0)),
                pl.BlockSpec((4, _D), lambda i: (0, 0))],
      out_specs=pl.BlockSpec((_R, _D), lambda i: (i, 0)),
      out_shape=jax.ShapeDtypeStruct((_N, _D), jnp.float32),
  )(x, a1)


def _t2_body(p_ref, x_ref, r1_ref, c1_ref, o_ref):
  agg = p_ref[0] + p_ref[1]
  z = agg + jnp.dot(x_ref[...], r1_ref[...],
                    preferred_element_type=jnp.float32) + c1_ref[...]
  o_ref[...] = jnp.maximum(z, 0.0)


def _tc_layer1(parts1, x, r1, c1):
  return pl.pallas_call(
      _t2_body,
      grid=(_NBLK,),
      in_specs=[pl.BlockSpec((2, _R, _D), lambda i: (0, i, 0)),
                pl.BlockSpec((_R, 4), lambda i: (i, 0)),
                pl.BlockSpec((4, _D), lambda i: (0, 0)),
                pl.BlockSpec((1, _D), lambda i: (0, 0))],
      out_specs=pl.BlockSpec((_R, _D), lambda i: (i, 0)),
      out_shape=jax.ShapeDtypeStruct((_N, _D), jnp.float32),
  )(parts1, x, r1, c1)


def _t3_body(p_ref, h1_ref, a2_ref, r2_ref, c2_ref, oa_ref, ob_ref):
  agg = p_ref[0] + p_ref[1]
  z = (jnp.dot(agg, a2_ref[...], preferred_element_type=jnp.float32)
       + jnp.dot(h1_ref[...], r2_ref[...], preferred_element_type=jnp.float32)
       + c2_ref[...])
  h2 = jnp.maximum(z, 0.0)
  oa_ref[...] = h2[:, :_D]
  ob_ref[...] = h2[:, _D:]


def _tc_layer2(parts2, h1, a2, r2, c2):
  return pl.pallas_call(
      _t3_body,
      grid=(_NBLK,),
      in_specs=[pl.BlockSpec((2, _R, _D), lambda i: (0, i, 0)),
                pl.BlockSpec((_R, _D), lambda i: (i, 0)),
                pl.BlockSpec((_D, 32), lambda i: (0, 0)),
                pl.BlockSpec((_D, 32), lambda i: (0, 0)),
                pl.BlockSpec((1, 32), lambda i: (0, 0))],
      out_specs=[pl.BlockSpec((_R, _D), lambda i: (i, 0)),
                 pl.BlockSpec((_R, _D), lambda i: (i, 0))],
      out_shape=[jax.ShapeDtypeStruct((_N, _D), jnp.float32),
                 jax.ShapeDtypeStruct((_N, _D), jnp.float32)],
  )(parts2, h1, a2, r2, c2)


def _t4_body(p_ref, ha_ref, hb_ref, a3_ref, r3_ref, c3_ref, b_ref,
             sums_ref, cnts_ref):
  agg = jnp.concatenate([p_ref[0], p_ref[1]], axis=1)
  h2 = jnp.concatenate([ha_ref[...], hb_ref[...]], axis=1)
  z = (jnp.dot(agg, a3_ref[...], preferred_element_type=jnp.float32)
       + jnp.dot(h2, r3_ref[...], preferred_element_type=jnp.float32)
       + c3_ref[...])
  h3 = jnp.maximum(z, 0.0)
  bb = b_ref[0, 0, :]
  oh = (bb[:, None] == lax.broadcasted_iota(jnp.int32, (_R, _G), 1)
        ).astype(jnp.float32)

  @pl.when(pl.program_id(0) == 0)
  def _():
    sums_ref[...] = jnp.zeros_like(sums_ref)
    cnts_ref[...] = jnp.zeros_like(cnts_ref)

  sums_ref[...] += lax.dot_general(oh, h3, (((0,), (0,)), ((), ())),
                                   preferred_element_type=jnp.float32)
  cnts_ref[...] += jnp.sum(oh, axis=0)[None, :]


def _tc_layer3_pool(parts3, h2a, h2b, a3, r3, c3, batch3):
  return pl.pallas_call(
      _t4_body,
      grid=(_NBLK,),
      in_specs=[pl.BlockSpec((2, _R, _D), lambda i: (0, i, 0)),
                pl.BlockSpec((_R, _D), lambda i: (i, 0)),
                pl.BlockSpec((_R, _D), lambda i: (i, 0)),
                pl.BlockSpec((32, 64), lambda i: (0, 0)),
                pl.BlockSpec((32, 64), lambda i: (0, 0)),
                pl.BlockSpec((1, 64), lambda i: (0, 0)),
                pl.BlockSpec((1, 1, _R), lambda i: (i, 0, 0))],
      out_specs=[pl.BlockSpec((_G, 64), lambda i: (0, 0)),
                 pl.BlockSpec((1, _G), lambda i: (0, 0))],
      out_shape=[jax.ShapeDtypeStruct((_G, 64), jnp.float32),
                 jax.ShapeDtypeStruct((1, _G), jnp.float32)],
  )(parts3, h2a, h2b, a3, r3, c3, batch3)


def _t5_body(sums_ref, cnts_ref, f1_ref, cf_ref, f4_ref, b4_ref, o_ref):
  cnt = jnp.maximum(cnts_ref[0, :], 1.0)
  pooled = sums_ref[...] / cnt[:, None]
  z = jnp.maximum(jnp.dot(pooled, f1_ref[...],
                          preferred_element_type=jnp.float32) + cf_ref[...],
                  0.0)
  o = jnp.dot(z, f4_ref[...], preferred_element_type=jnp.float32) + b4_ref[...]
  m = jnp.max(o, axis=1, keepdims=True)
  lse = m + jnp.log(jnp.sum(jnp.exp(o - m), axis=1, keepdims=True))
  o_ref[...] = o - lse


def _tc_head(sums, cnts, f1, cf, f4, b4):
  return pl.pallas_call(
      _t5_body,
      in_specs=[pl.BlockSpec((_G, 64), lambda: (0, 0)),
                pl.BlockSpec((1, _G), lambda: (0, 0)),
                pl.BlockSpec((64, _D), lambda: (0, 0)),
                pl.BlockSpec((1, _D), lambda: (0, 0)),
                pl.BlockSpec((_D, 2), lambda: (0, 0)),
                pl.BlockSpec((1, 2), lambda: (0, 0))],
      out_specs=pl.BlockSpec((_G, 2), lambda: (0, 0)),
      out_shape=jax.ShapeDtypeStruct((_G, 2), jnp.float32),
  )(sums, cnts, f1, cf, f4, b4)


def kernel(x, edge_index, edge_weight, edge_attr, batch,
           W1_rel, b1_rel, W1_root, W2_rel, b2_rel, W2_root,
           W3_rel, b3_rel, W3_root, Wfc1, bfc1, Wfc4, bfc4,
           bn1_g, bn1_b, bn1_rm, bn1_rv,
           bn2_g, bn2_b, bn2_rm, bn2_rv,
           bn3_g, bn3_b, bn3_rm, bn3_rv,
           bnf_g, bnf_b, bnf_rm, bnf_rv):
  eps = 1e-5
  s1 = bn1_g / jnp.sqrt(bn1_rv + eps)
  t1 = bn1_b - bn1_rm * s1
  s2 = bn2_g / jnp.sqrt(bn2_rv + eps)
  t2 = bn2_b - bn2_rm * s2
  s3 = bn3_g / jnp.sqrt(bn3_rv + eps)
  t3 = bn3_b - bn3_rm * s3
  sf = bnf_g / jnp.sqrt(bnf_rv + eps)
  tf = bnf_b - bnf_rm * sf

  a1 = W1_rel.T * s1[None, :]
  r1 = W1_root.T * s1[None, :]
  c1 = (b1_rel * s1 + t1)[None, :]
  a2 = W2_rel.T * s2[None, :]
  r2 = W2_root.T * s2[None, :]
  c2 = (b2_rel * s2 + t2)[None, :]
  a3 = W3_rel.T * s3[None, :]
  r3 = W3_root.T * s3[None, :]
  c3 = (b3_rel * s3 + t3)[None, :]
  f1 = Wfc1.T * sf[None, :]
  cf = (bfc1 * sf + tf)[None, :]
  f4 = Wfc4.T
  b4 = bfc4[None, :]

  edges3 = edge_index.reshape(2, _NCHUNK, _SG, _GRP)
  ew2 = edge_weight.reshape(_NCHUNK, _SG, _GRP)
  zeros = jnp.zeros((_STRIPE, _D), jnp.float32)
  batch3 = batch.reshape(_NBLK, 1, _R)

  y1 = _tc_pretable(x, a1)
  parts1 = _edge_pass_half(y1, y1, edges3, ew2, zeros)
  h1 = _tc_layer1(parts1, x, r1, c1)
  parts2 = _edge_pass_half(h1, h1, edges3, ew2, zeros)
  h2a, h2b = _tc_layer2(parts2, h1, a2, r2, c2)
  parts3 = _edge_pass_feat(h2a, h2b, edges3, ew2, zeros)
  sums, cnts = _tc_layer3_pool(parts3, h2a, h2b, a3, r3, c3, batch3)
  return _tc_head(sums, cnts, f1, cf, f4, b4)
